# Initial kernel scaffold; baseline (speedup 1.0000x reference)
import jax
import jax.numpy as jnp
from jax.experimental import pallas as pl


def kernel(x, weight):
    raise NotImplementedError("write your pallas kernel here")


# speedup vs baseline: 1.0938x; 1.0938x over previous
import functools

import jax
import jax.numpy as jnp
from jax import lax
from jax.experimental import pallas as pl
from jax.experimental.pallas import tpu as pltpu
from jax.experimental.pallas import tpu_sc as plsc

NC = 2
NS = 16
NW = NC * NS

D_EMB = 32

SUB = 128
K = 8
C = SUB * K


def _make_gather(B: int, D: int):
    assert B % (NW * C) == 0
    b_per_w = B // NW
    n_steps = b_per_w // C
    mesh = plsc.VectorSubcoreMesh(core_axis_name="c", subcore_axis_name="s")

    @functools.partial(
        pl.kernel,
        out_type=jax.ShapeDtypeStruct((B, D), jnp.float32),
        mesh=mesh,
        scratch_types=[
            pltpu.VMEM((C,), jnp.int32),
            pltpu.VMEM((C, D), jnp.float32),
            pltpu.SemaphoreType.DMA,
        ],
        compiler_params=pltpu.CompilerParams(use_tc_tiling_on_sc=False),
    )
    def gather_kernel(x_hbm, w_hbm, out_hbm, idx_v, rows_v, sem):
        wid = lax.axis_index("s") * NC + lax.axis_index("c")
        base = wid * b_per_w

        @pl.loop(0, n_steps)
        def _step(g):
            off = base + g * C
            pltpu.sync_copy(x_hbm.at[pl.ds(off, C)], idx_v)
            cps = [
                pltpu.async_copy(
                    w_hbm.at[idx_v.at[pl.ds(j * SUB, SUB)]],
                    rows_v.at[pl.ds(j * SUB, SUB)],
                    sem,
                )
                for j in range(K)
            ]
            for cp in cps:
                cp.wait()
            pltpu.sync_copy(rows_v, out_hbm.at[pl.ds(off, C)])

    return gather_kernel


def kernel(x, weight):
    xf = x.reshape(-1).astype(jnp.int32)
    out = _make_gather(xf.shape[0], weight.shape[1])(xf, weight)
    return out.reshape(x.shape + (weight.shape[1],))


# speedup vs baseline: 1.1122x; 1.0168x over previous
import functools

import jax
import jax.numpy as jnp
from jax import lax
from jax.experimental import pallas as pl
from jax.experimental.pallas import tpu as pltpu
from jax.experimental.pallas import tpu_sc as plsc

NC = 2
NS = 16
NW = NC * NS

D_EMB = 32

SUB = 128
K = 10
C = SUB * K


def _make_gather(B: int, D: int):
    assert B % (NW * C) == 0
    b_per_w = B // NW
    n_steps = b_per_w // C
    assert n_steps % 2 == 0
    mesh = plsc.VectorSubcoreMesh(core_axis_name="c", subcore_axis_name="s")

    @functools.partial(
        pl.kernel,
        out_type=jax.ShapeDtypeStruct((B, D), jnp.float32),
        mesh=mesh,
        scratch_types=[
            pltpu.VMEM((b_per_w,), jnp.int32),
            pltpu.VMEM((C, D), jnp.float32),
            pltpu.VMEM((C, D), jnp.float32),
            pltpu.SemaphoreType.DMA,
            pltpu.SemaphoreType.DMA,
            pltpu.SemaphoreType.DMA,
            pltpu.SemaphoreType.DMA,
        ],
        compiler_params=pltpu.CompilerParams(use_tc_tiling_on_sc=False),
    )
    def gather_kernel(x_hbm, w_hbm, out_hbm, idx_v, rows0, rows1,
                      gsem0, gsem1, wsem0, wsem1):
        rows = (rows0, rows1)
        gsem = (gsem0, gsem1)
        wsem = (wsem0, wsem1)
        wid = lax.axis_index("s") * NC + lax.axis_index("c")
        base = wid * b_per_w

        pltpu.sync_copy(x_hbm.at[pl.ds(base, b_per_w)], idx_v)

        def fire_gathers(g, b):
            for j in range(K):
                pltpu.async_copy(
                    w_hbm.at[idx_v.at[pl.ds(g * C + j * SUB, SUB)]],
                    rows[b].at[pl.ds(j * SUB, SUB)],
                    gsem[b],
                )

        def drain_gathers(b):
            pltpu.make_async_copy(w_hbm.at[pl.ds(0, C)], rows[b], gsem[b]).wait()

        def drain_writeback(b):
            pltpu.make_async_copy(
                rows[b], out_hbm.at[pl.ds(0, C)], wsem[b]).wait()

        fire_gathers(0, 0)

        @pl.loop(0, n_steps, step=2)
        def _step(g0):
            for b in range(2):
                g = g0 + b

                @pl.when(g >= 1)
                def _():
                    drain_writeback(1 - b)

                @pl.when(g + 1 < n_steps)
                def _():
                    fire_gathers(g + 1, 1 - b)

                drain_gathers(b)
                pltpu.async_copy(
                    rows[b], out_hbm.at[pl.ds(base + g * C, C)], wsem[b])

        drain_writeback((n_steps - 1) % 2)

    return gather_kernel


def kernel(x, weight):
    xf = x.reshape(-1).astype(jnp.int32)
    out = _make_gather(xf.shape[0], weight.shape[1])(xf, weight)
    return out.reshape(x.shape + (weight.shape[1],))


# speedup vs baseline: 1.5256x; 1.3717x over previous
import functools

import jax
import jax.numpy as jnp
from jax import lax
from jax.experimental import pallas as pl
from jax.experimental.pallas import tpu as pltpu
from jax.experimental.pallas import tpu_sc as plsc

NC = 2
NS = 16
NW = NC * NS

D_EMB = 32
LANES = 16


def _make_gather(n_tok: int, n_seq: int, D: int):
    assert n_tok % 128 == 0 and D % 8 == 0
    n_blocks = n_seq * (n_tok // 128)
    assert n_blocks % NW == 0
    blk_per_w = n_blocks // NW
    assert blk_per_w % 2 == 0
    out_rows = n_blocks * D
    mesh = plsc.VectorSubcoreMesh(core_axis_name="c", subcore_axis_name="s")

    @functools.partial(
        pl.kernel,
        out_type=jax.ShapeDtypeStruct((out_rows, 128), jnp.float32),
        mesh=mesh,
        scratch_types=[
            pltpu.VMEM((128,), jnp.int32),
            pltpu.VMEM((128,), jnp.int32),
            pltpu.VMEM((128, D), jnp.float32),
            pltpu.VMEM((128, D), jnp.float32),
            pltpu.VMEM((D, 128), jnp.float32),
            pltpu.VMEM((D, 128), jnp.float32),
            pltpu.SemaphoreType.DMA,
            pltpu.SemaphoreType.DMA,
            pltpu.SemaphoreType.DMA,
            pltpu.SemaphoreType.DMA,
            pltpu.SemaphoreType.DMA,
            pltpu.SemaphoreType.DMA,
        ],
        compiler_params=pltpu.CompilerParams(
            use_tc_tiling_on_sc=False, needs_layout_passes=False),
    )
    def gather_kernel(xt_hbm, w_hbm, out_hbm,
                      idx0, idx1, rows0, rows1, outb0, outb1,
                      isem0, isem1, gsem0, gsem1, wsem0, wsem1):
        idxb = (idx0, idx1)
        rows = (rows0, rows1)
        outb = (outb0, outb1)
        isem = (isem0, isem1)
        gsem = (gsem0, gsem1)
        wsem = (wsem0, wsem1)
        wid = lax.axis_index("s") * NC + lax.axis_index("c")
        blk0 = wid * blk_per_w

        def x_off(g):
            B = blk0 + g
            return pl.multiple_of(((B >> 7) << 14) + ((B & 127) << 7), 128)

        def fire_idx(g, b):
            pltpu.async_copy(xt_hbm.at[pl.ds(x_off(g), 128)], idxb[b], isem[b])

        def wait_idx(b):
            pltpu.make_async_copy(
                xt_hbm.at[pl.ds(0, 128)], idxb[b], isem[b]).wait()

        def fire_gather(b):
            pltpu.async_copy(w_hbm.at[idxb[b]], rows[b], gsem[b])

        def wait_gather(b):
            pltpu.make_async_copy(
                w_hbm.at[pl.ds(0, 128)], rows[b], gsem[b]).wait()

        def transpose(b):
            for d in range(D):
                for c in range(128 // LANES):
                    v = plsc.load_gather(
                        rows[b],
                        [lax.iota(jnp.int32, LANES) + (c * LANES),
                         jnp.full((LANES,), d, jnp.int32)],
                    )
                    outb[b][d, pl.ds(c * LANES, LANES)] = v

        def fire_writes(g, b):
            B = blk0 + g
            base = pl.multiple_of(((B >> 7) << 12) + ((B & 127) << 3), 8)
            for tr in range(D // 8):
                pltpu.async_copy(
                    outb[b].at[pl.ds(tr * 8, 8)],
                    out_hbm.at[pl.ds(base + tr * 1024, 8)],
                    wsem[b],
                )

        def wait_writes(b):
            pltpu.make_async_copy(
                outb[b], out_hbm.at[pl.ds(0, D)], wsem[b]).wait()

        fire_idx(0, 0)

        @pl.loop(0, blk_per_w, step=2)
        def _step(g0):
            for b in range(2):
                g = g0 + b
                wait_idx(b)

                @pl.when(g + 1 < blk_per_w)
                def _():
                    fire_idx(g + 1, 1 - b)

                @pl.when(g >= 2)
                def _():
                    wait_writes(b)

                fire_gather(b)

                @pl.when(g >= 1)
                def _():
                    wait_gather(1 - b)
                    transpose(1 - b)
                    fire_writes(g - 1, 1 - b)

        wait_gather(1)
        transpose(1)
        fire_writes(blk_per_w - 1, 1)
        wait_writes(0)
        wait_writes(1)

    return gather_kernel


def kernel(x, weight):
    n_tok, n_seq = x.shape
    D = weight.shape[1]
    xt = x.T.reshape(-1).astype(jnp.int32)
    out_k = _make_gather(n_tok, n_seq, D)(xt, weight)
    out = (
        out_k.reshape(n_seq, D // 8, n_tok // 128, 8, 128)
        .transpose(2, 4, 0, 1, 3)
        .reshape(n_tok, n_seq, D)
    )
    return out


# speedup vs baseline: 2.1693x; 1.4219x over previous
import functools

import jax
import jax.numpy as jnp
from jax import lax
from jax.experimental import pallas as pl
from jax.experimental.pallas import tpu as pltpu
from jax.experimental.pallas import tpu_sc as plsc

NC = 2
NS = 16
NW = NC * NS

D_EMB = 32
LANES = 16


def _make_gather(n_tok: int, n_seq: int, D: int):
    assert n_tok % 128 == 0 and D % 8 == 0
    n_blocks = n_seq * (n_tok // 128)
    assert n_blocks % NW == 0
    blk_per_w = n_blocks // NW
    assert blk_per_w % 2 == 0
    out_rows = n_blocks * D
    mesh = plsc.VectorSubcoreMesh(core_axis_name="c", subcore_axis_name="s")

    @functools.partial(
        pl.kernel,
        out_type=jax.ShapeDtypeStruct((out_rows, 128), jnp.float32),
        mesh=mesh,
        scratch_types=[
            pltpu.VMEM((128,), jnp.int32),
            pltpu.VMEM((128,), jnp.int32),
            pltpu.VMEM((128, D), jnp.float32),
            pltpu.VMEM((128, D), jnp.float32),
            pltpu.VMEM((D, 128), jnp.float32),
            pltpu.VMEM((D, 128), jnp.float32),
            pltpu.SemaphoreType.DMA,
            pltpu.SemaphoreType.DMA,
            pltpu.SemaphoreType.DMA,
            pltpu.SemaphoreType.DMA,
            pltpu.SemaphoreType.DMA,
            pltpu.SemaphoreType.DMA,
        ],
        compiler_params=pltpu.CompilerParams(
            use_tc_tiling_on_sc=False, needs_layout_passes=False,
            disable_bounds_checks=True),
    )
    def gather_kernel(xt_hbm, w_hbm, out_hbm,
                      idx0, idx1, rows0, rows1, outb0, outb1,
                      isem0, isem1, gsem0, gsem1, wsem0, wsem1):
        idxb = (idx0, idx1)
        rows = (rows0, rows1)
        outb = (outb0, outb1)
        isem = (isem0, isem1)
        gsem = (gsem0, gsem1)
        wsem = (wsem0, wsem1)
        wid = lax.axis_index("s") * NC + lax.axis_index("c")
        blk0 = wid * blk_per_w

        def x_off(g):
            B = blk0 + g
            return pl.multiple_of(((B >> 7) << 14) + ((B & 127) << 7), 128)

        def fire_idx(g, b):
            pltpu.async_copy(xt_hbm.at[pl.ds(x_off(g), 128)], idxb[b], isem[b])

        def wait_idx(b):
            pltpu.make_async_copy(
                xt_hbm.at[pl.ds(0, 128)], idxb[b], isem[b]).wait()

        def fire_gather(b):
            pltpu.async_copy(w_hbm.at[idxb[b]], rows[b], gsem[b])

        def wait_gather(b):
            pltpu.make_async_copy(
                w_hbm.at[pl.ds(0, 128)], rows[b], gsem[b]).wait()

        def transpose(b):
            @plsc.parallel_loop(0, D, unroll=4)
            def _t(d):
                for c in range(128 // LANES):
                    v = plsc.load_gather(
                        rows[b],
                        [lax.iota(jnp.int32, LANES) + (c * LANES),
                         jnp.full((LANES,), d, jnp.int32)],
                    )
                    outb[b][d, pl.ds(c * LANES, LANES)] = v

        def fire_writes(g, b):
            B = blk0 + g
            base = pl.multiple_of(((B >> 7) << 12) + ((B & 127) << 3), 8)
            for tr in range(D // 8):
                pltpu.async_copy(
                    outb[b].at[pl.ds(tr * 8, 8)],
                    out_hbm.at[pl.ds(base + tr * 1024, 8)],
                    wsem[b],
                )

        def wait_writes(b):
            pltpu.make_async_copy(
                outb[b], out_hbm.at[pl.ds(0, D)], wsem[b]).wait()

        fire_idx(0, 0)

        @pl.loop(0, blk_per_w, step=2)
        def _step(g0):
            for b in range(2):
                g = g0 + b
                wait_idx(b)

                @pl.when(g + 1 < blk_per_w)
                def _():
                    fire_idx(g + 1, 1 - b)

                @pl.when(g >= 2)
                def _():
                    wait_writes(b)

                fire_gather(b)

                @pl.when(g >= 1)
                def _():
                    wait_gather(1 - b)
                    transpose(1 - b)
                    fire_writes(g - 1, 1 - b)

        wait_gather(1)
        transpose(1)
        fire_writes(blk_per_w - 1, 1)
        wait_writes(0)
        wait_writes(1)

    return gather_kernel


def kernel(x, weight):
    n_tok, n_seq = x.shape
    D = weight.shape[1]
    xt = x.T.reshape(-1).astype(jnp.int32)
    out_k = _make_gather(n_tok, n_seq, D)(xt, weight)
    out = (
        out_k.reshape(n_seq, D // 8, n_tok // 128, 8, 128)
        .transpose(2, 4, 0, 1, 3)
        .reshape(n_tok, n_seq, D)
    )
    return out


# speedup vs baseline: 2.2122x; 1.0198x over previous
import functools

import jax
import jax.numpy as jnp
from jax import lax
from jax.experimental import pallas as pl
from jax.experimental.pallas import tpu as pltpu
from jax.experimental.pallas import tpu_sc as plsc

NC = 2
NS = 16
NW = NC * NS

LANES = 16
NBUF = 8
GLAG = 6


def _make_gather(n_tok: int, n_seq: int, D: int):
    assert n_tok % 128 == 0 and D % 8 == 0
    n_blocks = n_seq * (n_tok // 128)
    assert n_blocks % NW == 0
    nblk = n_blocks // NW
    out_rows = n_blocks * D
    n_iter = -(-(nblk + GLAG) // NBUF) * NBUF
    mesh = plsc.VectorSubcoreMesh(core_axis_name="c", subcore_axis_name="s")

    @functools.partial(
        pl.kernel,
        out_type=jax.ShapeDtypeStruct((out_rows, 128), jnp.float32),
        mesh=mesh,
        scratch_types=[
            pltpu.VMEM((nblk * 128,), jnp.int32),
            [pltpu.VMEM((128, D), jnp.float32) for _ in range(NBUF)],
            [pltpu.VMEM((D, 128), jnp.float32) for _ in range(2)],
            [pltpu.SemaphoreType.DMA for _ in range(NBUF)],
            [pltpu.SemaphoreType.DMA for _ in range(2)],
        ],
        compiler_params=pltpu.CompilerParams(
            use_tc_tiling_on_sc=False, needs_layout_passes=False,
            disable_bounds_checks=True),
    )
    def gather_kernel(xt_hbm, w_hbm, out_hbm,
                      idx_all, rows, outb, gsem, wsem):
        wid = lax.axis_index("s") * NC + lax.axis_index("c")
        blk0 = wid * nblk

        def fire_gather(g, b):
            pltpu.async_copy(
                w_hbm.at[idx_all.at[pl.ds(pl.multiple_of(g * 128, 128), 128)]],
                rows[b], gsem[b])

        def wait_gather(b):
            pltpu.make_async_copy(
                w_hbm.at[pl.ds(0, 128)], rows[b], gsem[b]).wait()

        def transpose(rb, ob):
            @plsc.parallel_loop(0, D, unroll=4)
            def _t(d):
                for c in range(128 // LANES):
                    v = plsc.load_gather(
                        rows[rb],
                        [lax.iota(jnp.int32, LANES) + (c * LANES),
                         jnp.full((LANES,), d, jnp.int32)],
                    )
                    outb[ob][d, pl.ds(c * LANES, LANES)] = v

        def fire_writes(k, ob):
            B = blk0 + k
            base = pl.multiple_of(((B >> 7) << 12) + ((B & 127) << 3), 8)
            for tr in range(D // 8):
                pltpu.async_copy(
                    outb[ob].at[pl.ds(tr * 8, 8)],
                    out_hbm.at[pl.ds(base + tr * 1024, 8)],
                    wsem[ob],
                )

        def wait_writes(ob):
            pltpu.make_async_copy(
                outb[ob], out_hbm.at[pl.ds(0, D)], wsem[ob]).wait()

        pltpu.sync_copy(xt_hbm.at[pl.ds(blk0 * 128, nblk * 128)], idx_all)

        @pl.loop(0, n_iter, step=NBUF)
        def _step(g0):
            for b in range(NBUF):
                g = g0 + b

                @pl.when(g < nblk)
                def _():
                    fire_gather(g, b)

                k = g - GLAG
                kb = (b + NBUF - GLAG) % NBUF

                @pl.when((k >= 0) & (k < nblk))
                def _():
                    wait_gather(kb)

                    @pl.when(k >= 2)
                    def _():
                        wait_writes((b + NBUF - GLAG) % 2)

                    transpose(kb, (b + NBUF - GLAG) % 2)
                    fire_writes(k, (b + NBUF - GLAG) % 2)

        wait_writes(0)
        wait_writes(1)

    return gather_kernel


def kernel(x, weight):
    n_tok, n_seq = x.shape
    D = weight.shape[1]
    xt = x.T.reshape(-1).astype(jnp.int32)
    out_k = _make_gather(n_tok, n_seq, D)(xt, weight)
    out = (
        out_k.reshape(n_seq, D // 8, n_tok // 128, 8, 128)
        .transpose(2, 4, 0, 1, 3)
        .reshape(n_tok, n_seq, D)
    )
    return out
